# baseline (speedup 1.0000x reference)
import functools

import jax
import jax.numpy as jnp
from jax import lax
from jax.experimental import pallas as pl
from jax.experimental.pallas import tpu as pltpu
from jax.experimental.pallas import tpu_sc as plsc

NC, NS, L = 2, 16, 16
NW = NC * NS

ROWS_PER_CHUNK = 4


@functools.lru_cache(maxsize=None)
def _gather_fn(B, S, D):
    n_b = B // NW
    n_chunks = n_b // ROWS_PER_CHUNK
    n_flat = n_b * S
    assert B == NW * n_b and n_b == n_chunks * ROWS_PER_CHUNK
    assert n_flat % L == 0 and (S % 8) == 0
    splits = []
    off = 0
    while off < S:
        g = min(128, S - off)
        splits.append((off, g))
        off += g
    mesh = plsc.VectorSubcoreMesh(core_axis_name="c", subcore_axis_name="s")

    @functools.partial(
        pl.kernel,
        mesh=mesh,
        compiler_params=pltpu.CompilerParams(use_tc_tiling_on_sc=False),
        out_type=jax.ShapeDtypeStruct((B, S, D), jnp.float32),
        scratch_types=[
            pltpu.VMEM((n_flat,), jnp.int32),
            pltpu.VMEM((ROWS_PER_CHUNK, S, 2 * D), jnp.float32),
            pltpu.VMEM((L,), jnp.int32),
            pltpu.SemaphoreType.DMA,
            pltpu.SemaphoreType.DMA,
        ],
    )
    def gather_kernel(table_hbm, idx_hbm, seqlen_hbm, out_hbm,
                      idx_v, rows_v, seql_v, sem, sem2):
        wid = lax.axis_index("s") * NC + lax.axis_index("c")
        b0 = wid * n_b
        stage = [pltpu.async_copy(idx_hbm.at[b0 + i],
                                  idx_v.at[pl.ds(i * S, S)], sem2)
                 for i in range(n_b)]
        pltpu.sync_copy(seqlen_hbm, seql_v)
        for h in stage:
            h.wait()
        seql = seql_v[...]
        lane = lax.iota(jnp.int32, L)

        def fix(k, carry):
            v = idx_v[pl.ds(k * L, L)]
            col = lax.rem(k * L + lane, S)
            idx_v[pl.ds(k * L, L)] = jnp.where(col < seql, v + 1,
                                               jnp.zeros_like(v))
            return carry

        lax.fori_loop(0, n_flat // L, fix, 0)

        for c in range(n_chunks):
            handles = []
            for i in range(ROWS_PER_CHUNK):
                flat0 = (c * ROWS_PER_CHUNK + i) * S
                for (off, g) in splits:
                    handles.append(pltpu.async_copy(
                        table_hbm.at[idx_v.at[pl.ds(flat0 + off, g)]],
                        rows_v.at[i, pl.ds(off, g)], sem))
            for h in handles:
                h.wait()
            pltpu.sync_copy(
                rows_v.at[:, :, pl.ds(0, D)],
                out_hbm.at[pl.ds(b0 + c * ROWS_PER_CHUNK, ROWS_PER_CHUNK)])

    return gather_kernel


def kernel(text, seq_len, text_embed_weight):
    B, S = text.shape
    D = text_embed_weight.shape[1]
    V = text_embed_weight.shape[0]
    VPAD = ((V + 7) // 8) * 8
    bounds = [0, 250112, 500224, 750336, V]
    slabs = []
    for i in range(4):
        sl = text_embed_weight[bounds[i]:bounds[i + 1]]
        rpad = 0 if i < 3 else (VPAD - bounds[3] - sl.shape[0])
        slabs.append(jnp.pad(sl, ((0, rpad), (0, D))))
    tpad = jnp.concatenate(slabs, axis=0)
    seql_vec = jnp.full((L,), seq_len, dtype=jnp.int32)
    return _gather_fn(B, S, D)(tpad, text, seql_vec)


# speedup vs baseline: 1.9545x; 1.9545x over previous
import functools

import jax
import jax.numpy as jnp
from jax import lax
from jax.experimental import pallas as pl
from jax.experimental.pallas import tpu as pltpu
from jax.experimental.pallas import tpu_sc as plsc

NC, NS, L = 2, 16, 16
NW = NC * NS

ROWS_PER_CHUNK = 4


@functools.lru_cache(maxsize=None)
def _gather_fn(B, S, D):
    n_b = B // NW
    n_chunks = n_b // ROWS_PER_CHUNK
    n_flat = n_b * S
    assert B == NW * n_b and n_b == n_chunks * ROWS_PER_CHUNK
    assert n_flat % L == 0 and (S % 8) == 0
    splits = []
    off = 0
    while off < S:
        g = min(128, S - off)
        splits.append((off, g))
        off += g
    mesh = plsc.VectorSubcoreMesh(core_axis_name="c", subcore_axis_name="s")

    @functools.partial(
        pl.kernel,
        mesh=mesh,
        compiler_params=pltpu.CompilerParams(use_tc_tiling_on_sc=False),
        out_type=jax.ShapeDtypeStruct((B, S, D), jnp.float32),
        scratch_types=[
            pltpu.VMEM((n_flat,), jnp.int32),
            pltpu.VMEM((ROWS_PER_CHUNK, S, 2 * D), jnp.float32),
            pltpu.VMEM((L,), jnp.int32),
            pltpu.SemaphoreType.DMA,
            pltpu.SemaphoreType.DMA,
        ],
    )
    def gather_kernel(table_hbm, idx_hbm, seqlen_hbm, out_hbm,
                      idx_v, rows_v, seql_v, sem, sem2):
        wid = lax.axis_index("s") * NC + lax.axis_index("c")
        b0 = wid * n_b
        stage = [pltpu.async_copy(idx_hbm.at[b0 + i],
                                  idx_v.at[pl.ds(i * S, S)], sem2)
                 for i in range(n_b)]
        pltpu.sync_copy(seqlen_hbm, seql_v)
        for h in stage:
            h.wait()
        seql = seql_v[...]
        lane = lax.iota(jnp.int32, L)

        def fix(k, carry):
            v = idx_v[pl.ds(k * L, L)]
            col = lax.rem(k * L + lane, S)
            idx_v[pl.ds(k * L, L)] = jnp.where(col < seql, v + 1,
                                               jnp.zeros_like(v))
            return carry

        lax.fori_loop(0, n_flat // L, fix, 0)

        for c in range(n_chunks):
            handles = []
            for i in range(ROWS_PER_CHUNK):
                flat0 = (c * ROWS_PER_CHUNK + i) * S
                for (off, g) in splits:
                    handles.append(pltpu.async_copy(
                        table_hbm.at[idx_v.at[pl.ds(flat0 + off, g)]],
                        rows_v.at[i, pl.ds(off, g)], sem))
            for h in handles:
                h.wait()
            pltpu.sync_copy(
                rows_v.at[:, :, pl.ds(0, D)],
                out_hbm.at[pl.ds(b0 + c * ROWS_PER_CHUNK, ROWS_PER_CHUNK)])

    return gather_kernel


def kernel(text, seq_len, text_embed_weight):
    B, S = text.shape
    D = text_embed_weight.shape[1]
    tpad = jnp.pad(text_embed_weight, ((0, 7), (0, D)))
    seql_vec = jnp.full((L,), seq_len, dtype=jnp.int32)
    return _gather_fn(B, S, D)(tpad, text, seql_vec)
